# baseline (speedup 1.0000x reference)
import functools

import jax
import jax.numpy as jnp
from jax import lax
from jax.experimental import pallas as pl
from jax.experimental.pallas import tpu as pltpu
from jax.experimental.pallas import tpu_sc as plsc

N = 10000
K = 32
D = 128
LAT = 64

NC = 2
NS = 16
NW = NC * NS
RPW = D // NW
NBLK = 512
NPAD = 10240
NB = NPAD // NBLK
INV_K = 1.0 / K


@functools.cache
def _build_gather_mean():
    mesh = plsc.VectorSubcoreMesh(core_axis_name="c", subcore_axis_name="s")
    return functools.partial(
        pl.kernel,
        mesh=mesh,
        compiler_params=pltpu.CompilerParams(needs_layout_passes=False),
        out_type=jax.ShapeDtypeStruct((D, NPAD), jnp.float32),
        scratch_types=[
            pltpu.VMEM((RPW // 2 * NPAD,), jnp.float32),
            pltpu.VMEM((2, K // 2, NBLK), jnp.int32),
            pltpu.VMEM((2, RPW, NBLK), jnp.float32),
            pltpu.SemaphoreType.DMA,
            pltpu.SemaphoreType.DMA,
            pltpu.SemaphoreType.DMA,
            pltpu.SemaphoreType.DMA,
        ],
    )(_gather_mean_body)


def _gather_mean(pt, adjt):
    return _build_gather_mean()(pt, adjt)


def _gather_mean_body(pt_hbm, adjt_hbm, out_hbm, tab_v, idx_v, stage_v, *sems):
    wid = lax.axis_index("s") * NC + lax.axis_index("c")
    row0 = wid * RPW

    for v in range(RPW // 2):
        pltpu.sync_copy(pt_hbm.at[wid * (RPW // 2) + v],
                        tab_v.at[pl.ds(v * NPAD, NPAD)])

    isems = sems[:2]
    osems = sems[2:]

    def start_idx(nb, b):
        pltpu.async_copy(
            adjt_hbm.at[pl.ds(0, K // 2), pl.ds(nb * NBLK, NBLK)], idx_v.at[b],
            isems[b])

    def wait_idx(b):
        pltpu.make_async_copy(
            adjt_hbm.at[pl.ds(0, K // 2), pl.ds(0, NBLK)], idx_v.at[b],
            isems[b]).wait()

    def start_out(nb, b):
        pltpu.async_copy(
            stage_v.at[b],
            out_hbm.at[pl.ds(row0, RPW), pl.ds(nb * NBLK, NBLK)], osems[b])

    def wait_out(b):
        pltpu.make_async_copy(
            stage_v.at[b], out_hbm.at[pl.ds(0, RPW), pl.ds(0, NBLK)], osems[b]).wait()

    def compute(b):
        def one_group(g16):
            lanes = pl.ds(g16 * 16, 16)
            accs = [jnp.zeros((16,), jnp.float32) for _ in range(RPW)]
            for j in range(K // 2):
                ipair = idx_v[b, j, lanes]
                ivecs = plsc.unpack(
                    plsc.bitcast(ipair, jnp.int16),
                    format=plsc.PackFormat.INTERLEAVED)
                for ivec in ivecs:
                    for v in range(RPW // 2):
                        flat = ivec + (v * NPAD) if v else ivec
                        packed = plsc.load_gather(tab_v, [flat])
                        even, odd = plsc.unpack(
                            plsc.bitcast(packed, jnp.bfloat16),
                            format=plsc.PackFormat.INTERLEAVED)
                        accs[2 * v] = accs[2 * v] + even
                        accs[2 * v + 1] = accs[2 * v + 1] + odd
            for u in range(RPW):
                stage_v[b, u, lanes] = accs[u] * INV_K

        def gbody(g, carry):
            one_group(2 * g)
            one_group(2 * g + 1)
            return carry

        lax.fori_loop(0, NBLK // 32, gbody, 0)

    start_idx(0, 0)
    start_idx(1, 1)

    def body(i, carry):
        for b in range(2):
            nb = 2 * i + b
            wait_idx(b)

            @pl.when(i > 0)
            def _():
                wait_out(b)

            compute(b)

            @pl.when(nb + 2 < NB)
            def _():
                start_idx(nb + 2, b)

            start_out(nb, b)
        return carry

    lax.fori_loop(0, NB // 2, body, 0)
    wait_out(0)
    wait_out(1)


_BLK = 2048
_GRID = NPAD // _BLK


def _col_spec(rows):
    return pl.BlockSpec((rows, _BLK), lambda i: (0, i))


def _full_spec(r, c):
    return pl.BlockSpec((r, c), lambda i: (0, 0))


def _dot_t(w, xt):
    return lax.dot_general(w.astype(jnp.bfloat16), xt.astype(jnp.bfloat16),
                           (((0,), (0,)), ((), ())),
                           preferred_element_type=jnp.float32)


def _pack_pairs(lo, hi):
    lo_u = lax.bitcast_convert_type(
        lax.convert_element_type(lo, jnp.bfloat16), jnp.uint16).astype(jnp.uint32)
    hi_u = lax.bitcast_convert_type(
        lax.convert_element_type(hi, jnp.bfloat16), jnp.uint16).astype(jnp.uint32)
    return lax.bitcast_convert_type((hi_u << 16) | lo_u, jnp.float32)


def _proj_packed(wn_e, wn_o, ht):
    return _pack_pairs(_dot_t(wn_e, ht), _dot_t(wn_o, ht))


def _tc_lift_body(xt_ref, cit_ref, wl_ref, wci_ref, wne_ref, wno_ref,
                  ht_ref, pt_ref):
    ht = _dot_t(wl_ref[...], xt_ref[...]) + _dot_t(wci_ref[...], cit_ref[...])
    ht_ref[...] = ht
    pt_ref[...] = _proj_packed(wne_ref[...], wno_ref[...], ht)


def _tc_lift(xt, cit, wl, wci, wne, wno):
    return pl.pallas_call(
        _tc_lift_body,
        grid=(_GRID,),
        in_specs=[_col_spec(1), _col_spec(2), _full_spec(1, D), _full_spec(2, D),
                  _full_spec(D, D // 2), _full_spec(D, D // 2)],
        out_specs=[_col_spec(D), _col_spec(D // 2)],
        out_shape=[jax.ShapeDtypeStruct((D, NPAD), jnp.float32),
                   jax.ShapeDtypeStruct((D // 2, NPAD), jnp.float32)],
    )(xt, cit, wl, wci, wne, wno)


def _tc_mp_body(ht_ref, mt_ref, ws_ref, wne_ref, wno_ref, ht_out_ref,
                pt_out_ref):
    hnt = jax.nn.gelu(_dot_t(ws_ref[...], ht_ref[...]) + mt_ref[...])
    ht_out_ref[...] = hnt
    pt_out_ref[...] = _proj_packed(wne_ref[...], wno_ref[...], hnt)


def _tc_mp(ht, mt, ws, wne, wno):
    return pl.pallas_call(
        _tc_mp_body,
        grid=(_GRID,),
        in_specs=[_col_spec(D), _col_spec(D), _full_spec(D, D),
                  _full_spec(D, D // 2), _full_spec(D, D // 2)],
        out_specs=[_col_spec(D), _col_spec(D // 2)],
        out_shape=[jax.ShapeDtypeStruct((D, NPAD), jnp.float32),
                   jax.ShapeDtypeStruct((D // 2, NPAD), jnp.float32)],
    )(ht, mt, ws, wne, wno)


def _tc_mid_body(ht_ref, mt_ref, ws_ref, wq_ref, wpost_ref, cot_ref, wco_ref,
                 epst_ref, wne_ref, wno_ref, momt_ref, gt_ref, pt_ref):
    h2t = jax.nn.gelu(_dot_t(ws_ref[...], ht_ref[...]) + mt_ref[...])
    momt = _dot_t(wq_ref[...], h2t)
    mut = momt[:LAT]
    logvart = jnp.clip(momt[LAT:], -30.0, 20.0)
    zt = mut + jnp.exp(0.5 * logvart) * epst_ref[...]
    gt = _dot_t(wpost_ref[...], zt) + _dot_t(wco_ref[...], cot_ref[...])
    momt_ref[...] = jnp.concatenate([mut, logvart], axis=0)
    gt_ref[...] = gt
    pt_ref[...] = _proj_packed(wne_ref[...], wno_ref[...], gt)


def _tc_mid(ht, mt, ws, wq, wpost, cot, wco, epst, wne, wno):
    return pl.pallas_call(
        _tc_mid_body,
        grid=(_GRID,),
        in_specs=[_col_spec(D), _col_spec(D), _full_spec(D, D),
                  _full_spec(D, 2 * LAT), _full_spec(LAT, D), _col_spec(2),
                  _full_spec(2, D), _col_spec(LAT), _full_spec(D, D // 2),
                  _full_spec(D, D // 2)],
        out_specs=[_col_spec(2 * LAT), _col_spec(D), _col_spec(D // 2)],
        out_shape=[jax.ShapeDtypeStruct((2 * LAT, NPAD), jnp.float32),
                   jax.ShapeDtypeStruct((D, NPAD), jnp.float32),
                   jax.ShapeDtypeStruct((D // 2, NPAD), jnp.float32)],
    )(ht, mt, ws, wq, wpost, cot, wco, epst, wne, wno)


def _tc_out_body(gt_ref, mt_ref, ws_ref, wout_ref, dect_ref):
    g2t = jax.nn.gelu(_dot_t(ws_ref[...], gt_ref[...]) + mt_ref[...])
    dect_ref[...] = jnp.sum(g2t * wout_ref[...], axis=0, keepdims=True)


def _tc_out(gt, mt, ws, wout):
    return pl.pallas_call(
        _tc_out_body,
        grid=(_GRID,),
        in_specs=[_col_spec(D), _col_spec(D), _full_spec(D, D), _full_spec(D, 1)],
        out_specs=[_col_spec(1)],
        out_shape=[jax.ShapeDtypeStruct((1, NPAD), jnp.float32)],
    )(gt, mt, ws, wout)[0]


def kernel(x, coords_input, coords_output, adjc, W_lift, W_coord_in, W_coord_out,
           W_es1, W_en1, W_es2, W_en2, W_q, W_post,
           W_ds1, W_dn1, W_ds2, W_dn2, W_out, eps):
    b = x.shape[0]
    pad = ((0, 0), (0, NPAD - N))
    xt = jnp.pad(x.reshape(1, N), pad)
    cit = jnp.pad(coords_input.T, pad)
    cot = jnp.pad(coords_output.T, pad)
    epst = jnp.pad(eps.T, pad)
    adjt_full = jnp.pad(adjc.T, pad)
    adjt = (adjt_full[1::2] << 16) | adjt_full[0::2]

    wn_eo = [(w[:, 0::2], w[:, 1::2]) for w in (W_en1, W_en2, W_dn1, W_dn2)]

    h0t, p0t = _tc_lift(xt, cit, W_lift, W_coord_in, *wn_eo[0])
    m1t = _gather_mean(p0t, adjt)
    h1t, p1t = _tc_mp(h0t, m1t, W_es1, *wn_eo[1])
    m2t = _gather_mean(p1t, adjt)
    momt, g0t, p2t = _tc_mid(h1t, m2t, W_es2, W_q, W_post, cot, W_coord_out,
                             epst, *wn_eo[2])
    m3t = _gather_mean(p2t, adjt)
    g1t, p3t = _tc_mp(g0t, m3t, W_ds1, *wn_eo[3])
    m4t = _gather_mean(p3t, adjt)
    dect = _tc_out(g1t, m4t, W_ds2, W_out)

    return (dect[:, :N].reshape(b, N, 1), momt[:LAT, :N].T,
            momt[LAT:, :N].T)


# speedup vs baseline: 1.0017x; 1.0017x over previous
import functools

import jax
import jax.numpy as jnp
from jax import lax
from jax.experimental import pallas as pl
from jax.experimental.pallas import tpu as pltpu
from jax.experimental.pallas import tpu_sc as plsc

N = 10000
K = 32
D = 128
LAT = 64

NC = 2
NS = 16
NW = NC * NS
RPW = D // NW
NBLK = 512
NPAD = 10240
NB = NPAD // NBLK
INV_K = 1.0 / K


@functools.cache
def _build_gather_mean():
    mesh = plsc.VectorSubcoreMesh(core_axis_name="c", subcore_axis_name="s")
    return functools.partial(
        pl.kernel,
        mesh=mesh,
        compiler_params=pltpu.CompilerParams(needs_layout_passes=False),
        out_type=jax.ShapeDtypeStruct((D, NPAD), jnp.float32),
        scratch_types=[
            pltpu.VMEM((RPW // 2 * NPAD,), jnp.float32),
            pltpu.VMEM((2, K // 2, NBLK), jnp.int32),
            pltpu.VMEM((2, RPW, NBLK), jnp.float32),
            pltpu.SemaphoreType.DMA,
            pltpu.SemaphoreType.DMA,
            pltpu.SemaphoreType.DMA,
            pltpu.SemaphoreType.DMA,
        ],
    )(_gather_mean_body)


def _gather_mean(pt, adjt):
    return _build_gather_mean()(pt, adjt)


def _gather_mean_body(pt_hbm, adjt_hbm, out_hbm, tab_v, idx_v, stage_v, *sems):
    wid = lax.axis_index("s") * NC + lax.axis_index("c")
    row0 = wid * RPW

    isems = sems[:2]
    osems = sems[2:]

    for v in range(RPW // 2):
        pltpu.async_copy(pt_hbm.at[wid * (RPW // 2) + v],
                         tab_v.at[pl.ds(v * NPAD, NPAD)], osems[v])

    def start_idx(nb, b):
        pltpu.async_copy(
            adjt_hbm.at[pl.ds(0, K // 2), pl.ds(nb * NBLK, NBLK)], idx_v.at[b],
            isems[b])

    def wait_idx(b):
        pltpu.make_async_copy(
            adjt_hbm.at[pl.ds(0, K // 2), pl.ds(0, NBLK)], idx_v.at[b],
            isems[b]).wait()

    def start_out(nb, b):
        pltpu.async_copy(
            stage_v.at[b],
            out_hbm.at[pl.ds(row0, RPW), pl.ds(nb * NBLK, NBLK)], osems[b])

    def wait_out(b):
        pltpu.make_async_copy(
            stage_v.at[b], out_hbm.at[pl.ds(0, RPW), pl.ds(0, NBLK)], osems[b]).wait()

    def compute(b):
        def one_group(g16):
            lanes = pl.ds(g16 * 16, 16)
            accs = [jnp.zeros((16,), jnp.float32) for _ in range(RPW)]
            for j in range(K // 2):
                ipair = idx_v[b, j, lanes]
                ivecs = plsc.unpack(
                    plsc.bitcast(ipair, jnp.int16),
                    format=plsc.PackFormat.INTERLEAVED)
                for ivec in ivecs:
                    for v in range(RPW // 2):
                        flat = ivec + (v * NPAD) if v else ivec
                        packed = plsc.load_gather(tab_v, [flat])
                        even, odd = plsc.unpack(
                            plsc.bitcast(packed, jnp.bfloat16),
                            format=plsc.PackFormat.INTERLEAVED)
                        accs[2 * v] = accs[2 * v] + even
                        accs[2 * v + 1] = accs[2 * v + 1] + odd
            for u in range(RPW):
                stage_v[b, u, lanes] = accs[u] * INV_K

        def gbody(g, carry):
            for t in range(4):
                one_group(4 * g + t)
            return carry

        lax.fori_loop(0, NBLK // 64, gbody, 0)

    start_idx(0, 0)
    start_idx(1, 1)
    for v in range(RPW // 2):
        pltpu.make_async_copy(pt_hbm.at[0], tab_v.at[pl.ds(0, NPAD)],
                              osems[v]).wait()

    def body(i, carry):
        for b in range(2):
            nb = 2 * i + b
            wait_idx(b)

            @pl.when(i > 0)
            def _():
                wait_out(b)

            compute(b)

            @pl.when(nb + 2 < NB)
            def _():
                start_idx(nb + 2, b)

            start_out(nb, b)
        return carry

    lax.fori_loop(0, NB // 2, body, 0)
    wait_out(0)
    wait_out(1)


_BLK = 2048
_GRID = NPAD // _BLK


def _col_spec(rows):
    return pl.BlockSpec((rows, _BLK), lambda i: (0, i))


def _full_spec(r, c):
    return pl.BlockSpec((r, c), lambda i: (0, 0))


def _dot_t(w, xt):
    return lax.dot_general(w, xt, (((0,), (0,)), ((), ())),
                           preferred_element_type=jnp.float32)


def _pack_pairs(lo, hi):
    lo_u = lax.bitcast_convert_type(
        lax.convert_element_type(lo, jnp.bfloat16), jnp.uint16).astype(jnp.uint32)
    hi_u = lax.bitcast_convert_type(
        lax.convert_element_type(hi, jnp.bfloat16), jnp.uint16).astype(jnp.uint32)
    return lax.bitcast_convert_type((hi_u << 16) | lo_u, jnp.float32)


def _proj_packed(wn_e, wn_o, ht):
    return _pack_pairs(_dot_t(wn_e, ht), _dot_t(wn_o, ht))


def _tc_lift_body(xt_ref, cit_ref, wl_ref, wci_ref, wne_ref, wno_ref,
                  ht_ref, pt_ref):
    ht = _dot_t(wl_ref[...], xt_ref[...]) + _dot_t(wci_ref[...], cit_ref[...])
    ht_ref[...] = ht
    pt_ref[...] = _proj_packed(wne_ref[...], wno_ref[...], ht)


def _tc_lift(xt, cit, wl, wci, wne, wno):
    return pl.pallas_call(
        _tc_lift_body,
        grid=(_GRID,),
        in_specs=[_col_spec(1), _col_spec(2), _full_spec(1, D), _full_spec(2, D),
                  _full_spec(D, D // 2), _full_spec(D, D // 2)],
        out_specs=[_col_spec(D), _col_spec(D // 2)],
        out_shape=[jax.ShapeDtypeStruct((D, NPAD), jnp.float32),
                   jax.ShapeDtypeStruct((D // 2, NPAD), jnp.float32)],
    )(xt, cit, wl, wci, wne, wno)


def _tc_mp_body(ht_ref, mt_ref, ws_ref, wne_ref, wno_ref, ht_out_ref,
                pt_out_ref):
    hnt = jax.nn.gelu(_dot_t(ws_ref[...], ht_ref[...]) + mt_ref[...])
    ht_out_ref[...] = hnt
    pt_out_ref[...] = _proj_packed(wne_ref[...], wno_ref[...], hnt)


def _tc_mp(ht, mt, ws, wne, wno):
    return pl.pallas_call(
        _tc_mp_body,
        grid=(_GRID,),
        in_specs=[_col_spec(D), _col_spec(D), _full_spec(D, D),
                  _full_spec(D, D // 2), _full_spec(D, D // 2)],
        out_specs=[_col_spec(D), _col_spec(D // 2)],
        out_shape=[jax.ShapeDtypeStruct((D, NPAD), jnp.float32),
                   jax.ShapeDtypeStruct((D // 2, NPAD), jnp.float32)],
    )(ht, mt, ws, wne, wno)


def _tc_mid_body(ht_ref, mt_ref, ws_ref, wq_ref, wpost_ref, cot_ref, wco_ref,
                 epst_ref, wne_ref, wno_ref, momt_ref, gt_ref, pt_ref):
    h2t = jax.nn.gelu(_dot_t(ws_ref[...], ht_ref[...]) + mt_ref[...])
    momt = _dot_t(wq_ref[...], h2t)
    mut = momt[:LAT]
    logvart = jnp.clip(momt[LAT:], -30.0, 20.0)
    zt = mut + jnp.exp(0.5 * logvart) * epst_ref[...]
    gt = _dot_t(wpost_ref[...], zt) + _dot_t(wco_ref[...], cot_ref[...])
    momt_ref[...] = jnp.concatenate([mut, logvart], axis=0)
    gt_ref[...] = gt
    pt_ref[...] = _proj_packed(wne_ref[...], wno_ref[...], gt)


def _tc_mid(ht, mt, ws, wq, wpost, cot, wco, epst, wne, wno):
    return pl.pallas_call(
        _tc_mid_body,
        grid=(_GRID,),
        in_specs=[_col_spec(D), _col_spec(D), _full_spec(D, D),
                  _full_spec(D, 2 * LAT), _full_spec(LAT, D), _col_spec(2),
                  _full_spec(2, D), _col_spec(LAT), _full_spec(D, D // 2),
                  _full_spec(D, D // 2)],
        out_specs=[_col_spec(2 * LAT), _col_spec(D), _col_spec(D // 2)],
        out_shape=[jax.ShapeDtypeStruct((2 * LAT, NPAD), jnp.float32),
                   jax.ShapeDtypeStruct((D, NPAD), jnp.float32),
                   jax.ShapeDtypeStruct((D // 2, NPAD), jnp.float32)],
    )(ht, mt, ws, wq, wpost, cot, wco, epst, wne, wno)


def _tc_out_body(gt_ref, mt_ref, ws_ref, wout_ref, dect_ref):
    g2t = jax.nn.gelu(_dot_t(ws_ref[...], gt_ref[...]) + mt_ref[...])
    dect_ref[...] = jnp.sum(g2t * wout_ref[...], axis=0, keepdims=True)


def _tc_out(gt, mt, ws, wout):
    return pl.pallas_call(
        _tc_out_body,
        grid=(_GRID,),
        in_specs=[_col_spec(D), _col_spec(D), _full_spec(D, D), _full_spec(D, 1)],
        out_specs=[_col_spec(1)],
        out_shape=[jax.ShapeDtypeStruct((1, NPAD), jnp.float32)],
    )(gt, mt, ws, wout)[0]


def kernel(x, coords_input, coords_output, adjc, W_lift, W_coord_in, W_coord_out,
           W_es1, W_en1, W_es2, W_en2, W_q, W_post,
           W_ds1, W_dn1, W_ds2, W_dn2, W_out, eps):
    b = x.shape[0]
    pad = ((0, 0), (0, NPAD - N))
    xt = jnp.pad(x.reshape(1, N), pad)
    cit = jnp.pad(coords_input.T, pad)
    cot = jnp.pad(coords_output.T, pad)
    epst = jnp.pad(eps.T, pad)
    adjt_full = jnp.pad(adjc.T, pad)
    adjt = (adjt_full[1::2] << 16) | adjt_full[0::2]

    wn_eo = [(w[:, 0::2], w[:, 1::2]) for w in (W_en1, W_en2, W_dn1, W_dn2)]

    h0t, p0t = _tc_lift(xt, cit, W_lift, W_coord_in, *wn_eo[0])
    m1t = _gather_mean(p0t, adjt)
    h1t, p1t = _tc_mp(h0t, m1t, W_es1, *wn_eo[1])
    m2t = _gather_mean(p1t, adjt)
    momt, g0t, p2t = _tc_mid(h1t, m2t, W_es2, W_q, W_post, cot, W_coord_out,
                             epst, *wn_eo[2])
    m3t = _gather_mean(p2t, adjt)
    g1t, p3t = _tc_mp(g0t, m3t, W_ds1, *wn_eo[3])
    m4t = _gather_mean(p3t, adjt)
    dect = _tc_out(g1t, m4t, W_ds2, W_out)

    return (dect[:, :N].reshape(b, N, 1), momt[:LAT, :N].T,
            momt[LAT:, :N].T)
